# baseline (speedup 1.0000x reference)
import functools

import jax
import jax.numpy as jnp
from jax.experimental import pallas as pl
from jax.experimental.pallas import tpu as pltpu

DIM = 256
N_EMBED = 1024
N_TOK = 8192
TB = 1024
HW = 1024
NB = N_TOK // TB
COMMITMENT_COST = 0.25


def _vq_block(z_ref, e_ref, esq_ref, enc_ref, zq_ref, idx_ref, loss_ref,
              perp_ref, counts_ref, acc_ref):
    i = pl.program_id(0)
    zb = z_ref[...]
    ew = e_ref[...]

    @pl.when(i == 0)
    def _init():
        acc_ref[0] = 0.0
        counts_ref[...] = jnp.zeros_like(counts_ref)

    zsq = jnp.sum(zb * zb, axis=1, keepdims=True)
    m2 = jax.lax.dot_general(zb * -2.0, ew, (((1,), (1,)), ((), ())),
                             preferred_element_type=jnp.float32)
    d = (zsq + esq_ref[...]) + m2

    rowmin = jnp.min(d, axis=1, keepdims=True)
    iota = jax.lax.broadcasted_iota(
        jnp.int32, (TB, N_EMBED), 1).astype(jnp.float32)
    idx_f = jnp.min(jnp.where(d == rowmin, iota, jnp.float32(N_EMBED)),
                    axis=1, keepdims=True)
    onehot = (iota == idx_f).astype(jnp.float32)

    enc_ref[...] = onehot
    idx_ref[0, 0, :] = idx_f[:, 0].astype(jnp.int32)

    zq = jax.lax.dot_general(onehot.astype(jnp.bfloat16),
                             ew.astype(jnp.bfloat16),
                             (((1,), (0,)), ((), ())),
                             preferred_element_type=jnp.float32)
    zq_ref[0] = jnp.transpose(zb + (zq - zb))

    acc_ref[0] += jnp.sum(rowmin)
    new_counts = counts_ref[...] + jnp.sum(onehot, axis=0, keepdims=True)
    counts_ref[...] = new_counts

    @pl.when(i == NB - 1)
    def _finalize():
        mse = acc_ref[0] / jnp.float32(N_TOK * DIM)
        loss_ref[...] = jnp.reshape(mse + COMMITMENT_COST * mse, (1, 1))
        p = new_counts * jnp.float32(1.0 / N_TOK)
        perp_ref[...] = jnp.reshape(
            jnp.exp(-jnp.sum(p * jnp.log(p + 1e-10))), (1, 1))


@functools.partial(jax.jit, static_argnums=())
def kernel(z, embed_weight):
    b, c, h, w = z.shape
    z_flat = jnp.transpose(z, (0, 2, 3, 1)).reshape(-1, DIM)
    esq = jnp.sum(embed_weight ** 2, axis=1)[None, :]

    enc, zq_st, idx3, loss, perp = pl.pallas_call(
        _vq_block,
        grid=(NB,),
        in_specs=[
            pl.BlockSpec((TB, DIM), lambda i: (i, 0)),
            pl.BlockSpec((N_EMBED, DIM), lambda i: (0, 0)),
            pl.BlockSpec((1, N_EMBED), lambda i: (0, 0)),
        ],
        out_specs=[
            pl.BlockSpec((TB, N_EMBED), lambda i: (i, 0)),
            pl.BlockSpec((1, DIM, HW), lambda i: (i, 0, 0)),
            pl.BlockSpec((1, 1, TB), lambda i: (i, 0, 0)),
            pl.BlockSpec((1, 1), lambda i: (0, 0)),
            pl.BlockSpec((1, 1), lambda i: (0, 0)),
        ],
        out_shape=[
            jax.ShapeDtypeStruct((N_TOK, N_EMBED), jnp.float32),
            jax.ShapeDtypeStruct((8, DIM, HW), jnp.float32),
            jax.ShapeDtypeStruct((NB, 1, TB), jnp.int32),
            jax.ShapeDtypeStruct((1, 1), jnp.float32),
            jax.ShapeDtypeStruct((1, 1), jnp.float32),
        ],
        scratch_shapes=[
            pltpu.VMEM((1, N_EMBED), jnp.float32),
            pltpu.SMEM((1,), jnp.float32),
        ],
    )(z_flat, embed_weight, esq)

    z_q_out = zq_st.reshape(b, c, h, w)
    return (loss.reshape(()), z_q_out, perp.reshape(()), enc,
            idx3.reshape(N_TOK, 1))


# speedup vs baseline: 1.2840x; 1.2840x over previous
import functools

import jax
import jax.numpy as jnp
from jax.experimental import pallas as pl
from jax.experimental.pallas import tpu as pltpu

DIM = 256
N_EMBED = 1024
N_TOK = 8192
TB = 2048
NB = N_TOK // TB
COMMITMENT_COST = 0.25


def _vq_block(z_ref, e_ref, esq_ref, enc_ref, zq_ref, idx_ref, loss_ref,
              perp_ref, counts_ref, acc_ref):
    i = pl.program_id(0)
    zb = z_ref[...]
    ew = e_ref[...]

    @pl.when(i == 0)
    def _init():
        acc_ref[0] = 0.0
        counts_ref[...] = jnp.zeros_like(counts_ref)

    zsq = jnp.sum(zb * zb, axis=1, keepdims=True)
    m2 = jax.lax.dot_general(zb * -2.0, ew, (((1,), (1,)), ((), ())),
                             preferred_element_type=jnp.float32)
    d = (zsq + esq_ref[...]) + m2

    rowmin = jnp.min(d, axis=1, keepdims=True)
    iota = jax.lax.broadcasted_iota(
        jnp.int32, (TB, N_EMBED), 1).astype(jnp.float32)
    idx_f = jnp.min(jnp.where(d == rowmin, iota, jnp.float32(N_EMBED)),
                    axis=1, keepdims=True)
    onehot = (iota == idx_f).astype(jnp.float32)

    enc_ref[...] = onehot
    idx_ref[0, 0, :] = idx_f[:, 0].astype(jnp.int32)

    zq = jax.lax.dot_general(onehot.astype(jnp.bfloat16),
                             ew.astype(jnp.bfloat16),
                             (((1,), (0,)), ((), ())),
                             preferred_element_type=jnp.float32)
    zq_ref[...] = zb + (zq - zb)

    acc_ref[0] += jnp.sum(rowmin)
    new_counts = counts_ref[...] + jnp.sum(onehot, axis=0, keepdims=True)
    counts_ref[...] = new_counts

    @pl.when(i == NB - 1)
    def _finalize():
        mse = acc_ref[0] / jnp.float32(N_TOK * DIM)
        loss_ref[...] = jnp.reshape(mse + COMMITMENT_COST * mse, (1, 1))
        p = new_counts * jnp.float32(1.0 / N_TOK)
        perp_ref[...] = jnp.reshape(
            jnp.exp(-jnp.sum(p * jnp.log(p + 1e-10))), (1, 1))


@functools.partial(jax.jit, static_argnums=())
def kernel(z, embed_weight):
    b, c, h, w = z.shape
    z_flat = jnp.transpose(z, (0, 2, 3, 1)).reshape(-1, DIM)
    esq = jnp.sum(embed_weight ** 2, axis=1)[None, :]

    enc, zq_st, idx3, loss, perp = pl.pallas_call(
        _vq_block,
        grid=(NB,),
        in_specs=[
            pl.BlockSpec((TB, DIM), lambda i: (i, 0)),
            pl.BlockSpec((N_EMBED, DIM), lambda i: (0, 0)),
            pl.BlockSpec((1, N_EMBED), lambda i: (0, 0)),
        ],
        out_specs=[
            pl.BlockSpec((TB, N_EMBED), lambda i: (i, 0)),
            pl.BlockSpec((TB, DIM), lambda i: (i, 0)),
            pl.BlockSpec((1, 1, TB), lambda i: (i, 0, 0)),
            pl.BlockSpec((1, 1), lambda i: (0, 0)),
            pl.BlockSpec((1, 1), lambda i: (0, 0)),
        ],
        out_shape=[
            jax.ShapeDtypeStruct((N_TOK, N_EMBED), jnp.float32),
            jax.ShapeDtypeStruct((N_TOK, DIM), jnp.float32),
            jax.ShapeDtypeStruct((NB, 1, TB), jnp.int32),
            jax.ShapeDtypeStruct((1, 1), jnp.float32),
            jax.ShapeDtypeStruct((1, 1), jnp.float32),
        ],
        scratch_shapes=[
            pltpu.VMEM((1, N_EMBED), jnp.float32),
            pltpu.SMEM((1,), jnp.float32),
        ],
    )(z_flat, embed_weight, esq)

    z_q_out = jnp.transpose(zq_st.reshape(b, h, w, c), (0, 3, 1, 2))
    return (loss.reshape(()), z_q_out, perp.reshape(()), enc,
            idx3.reshape(N_TOK, 1))
